# Initial kernel scaffold; baseline (speedup 1.0000x reference)
import jax
import jax.numpy as jnp
from jax.experimental import pallas as pl


def kernel(x, edge_index, edge_attr, batch, params):
    raise NotImplementedError("write your pallas kernel here")


# speedup vs baseline: 1.3036x; 1.3036x over previous
import functools
import jax
import jax.numpy as jnp
from jax.experimental import pallas as pl
from jax.experimental.pallas import tpu as pltpu

N = 10000
G = 64
H = 256
OUT = 64


def _leaky(v):
    return jnp.where(v > 0, v, 0.01 * v)


def _elu(v):
    return jnp.where(v > 0, v, jnp.expm1(v))


def _gru(inp, hid, Wih, Whh, bih, bhh):
    gi = inp @ Wih.T + bih
    gh = hid @ Whh.T + bhh
    ir, iz, inn = jnp.split(gi, 3, -1)
    hr, hz, hn = jnp.split(gh, 3, -1)
    r = jax.nn.sigmoid(ir + hr)
    z = jax.nn.sigmoid(iz + hz)
    n = jnp.tanh(inn + r * hn)
    return (1 - z) * n + z * hid


def _predict_body(out_ref, w_ref, b_ref, o_ref):
    o_ref[...] = out_ref[...] @ w_ref[...].T + b_ref[...]


def kernel(x, edge_index, edge_attr, batch, params):
    p = params
    src, dst = edge_index[0], edge_index[1]

    x1 = _leaky(x @ p['lin1_W'].T + p['lin1_b'])
    W1a = p['g_lin1_W'][:, :H]
    W1b = p['g_lin1_W'][:, H:]
    xa = x1 @ W1a.T
    eaW = edge_attr @ W1b.T
    alpha_j = (_leaky(xa[src] + eaW) * p['g_att_l']).sum(-1)
    alpha_i = x1 @ p['g_att_r']
    alpha = _leaky(alpha_j + alpha_i[dst])
    ex = jnp.exp(alpha)
    s = jax.ops.segment_sum(ex, dst, num_segments=N)
    w = ex / (s[dst] + 1e-16)
    xw2 = x1 @ p['g_lin2_W'].T
    h = jax.ops.segment_sum(xw2[src] * w[:, None], dst, num_segments=N) + p['g_bias']
    xc = jax.nn.relu(_gru(_elu(h), x1, p['gru0_Wih'], p['gru0_Whh'], p['gru0_bih'], p['gru0_bhh']))
    for cname, gname in [('conv1', 'gru1'), ('conv2', 'gru2')]:
        xt = xc @ p[cname + '_W'].T
        ss = xt @ p[cname + '_att_src']
        sd = xt @ p[cname + '_att_dst']
        a = _leaky(ss[src] + sd[dst])
        ex2 = jnp.exp(a)
        s2 = jax.ops.segment_sum(ex2, dst, num_segments=N)
        w2 = ex2 / (s2[dst] + 1e-16)
        h2 = jax.ops.segment_sum(xt[src] * w2[:, None], dst, num_segments=N) + p[cname + '_bias']
        xc = jax.nn.relu(_gru(_elu(h2), xc, p[gname + '_Wih'], p[gname + '_Whh'],
                              p[gname + '_bih'], p[gname + '_bhh']))
    onehot = (batch[:, None] == jnp.arange(G)[None, :]).astype(jnp.float32)
    out = jax.nn.relu(onehot.T @ xc)
    xt = xc @ p['mol_W'].T
    a_src = xt @ p['mol_att_src']
    for _ in range(2):
        ot = out @ p['mol_W'].T
        ad = ot @ p['mol_att_dst']
        a = _leaky(a_src + onehot @ ad)
        ex3 = jnp.exp(a)
        s3 = onehot @ (onehot.T @ ex3)
        w3 = ex3 / (s3 + 1e-16)
        hm = _elu((onehot * w3[:, None]).T @ xt + p['mol_bias'])
        out = jax.nn.relu(_gru(hm, out, p['mol_gru_Wih'], p['mol_gru_Whh'],
                               p['mol_gru_bih'], p['mol_gru_bhh']))

    return pl.pallas_call(
        _predict_body,
        out_shape=jax.ShapeDtypeStruct((G, OUT), jnp.float32),
    )(out, p['lin2_W'], p['lin2_b'])


# speedup vs baseline: 7.3055x; 5.6041x over previous
import functools
import math

import jax
import jax.numpy as jnp
from jax import lax
from jax.experimental import pallas as pl
from jax.experimental.pallas import tpu as pltpu
from jax.experimental.pallas import tpu_sc as plsc

N = 10000
E = 320000
G = 64
IN = 128
H = 256
OUT = 64
NS = 16
NCORE = 2
C = 80
NB = 1000

_SC_PARAMS = pltpu.CompilerParams(needs_layout_passes=False)


def _leaky(v):
    return jnp.where(v > 0, v, 0.01 * v)


def _elu(v):
    return jnp.where(v > 0, v, jnp.exp(jnp.minimum(v, 0.0)) - 1.0)


def _tc_pre_body(x_r, w1_r, b1_r, wa_r, atr_r, w2_r, x1_r, xa_r, xw2_r, ai_r):
    x1 = _leaky(jnp.dot(x_r[...], w1_r[...].T, preferred_element_type=jnp.float32)
                + b1_r[...][None, :])
    x1_r[...] = x1
    xa_r[...] = jnp.dot(x1, wa_r[...].T, preferred_element_type=jnp.float32)
    xw2_r[...] = jnp.dot(x1, w2_r[...].T, preferred_element_type=jnp.float32)
    ai_r[...] = jnp.sum(x1 * atr_r[...][None, :], axis=-1, keepdims=True)


def _tc_pre(x, w1, b1, wa, atr, w2):
    grid = (N // NB,)
    return pl.pallas_call(
        _tc_pre_body,
        grid=grid,
        in_specs=[
            pl.BlockSpec((NB, IN), lambda i: (i, 0)),
            pl.BlockSpec((H, IN), lambda i: (0, 0)),
            pl.BlockSpec((H,), lambda i: (0,)),
            pl.BlockSpec((H, H), lambda i: (0, 0)),
            pl.BlockSpec((H,), lambda i: (0,)),
            pl.BlockSpec((H, H), lambda i: (0, 0)),
        ],
        out_specs=[
            pl.BlockSpec((NB, H), lambda i: (i, 0)),
            pl.BlockSpec((NB, H), lambda i: (i, 0)),
            pl.BlockSpec((NB, H), lambda i: (i, 0)),
            pl.BlockSpec((NB, 1), lambda i: (i, 0)),
        ],
        out_shape=[
            jax.ShapeDtypeStruct((N, H), jnp.float32),
            jax.ShapeDtypeStruct((N, H), jnp.float32),
            jax.ShapeDtypeStruct((N, H), jnp.float32),
            jax.ShapeDtypeStruct((N, 1), jnp.float32),
        ],
    )(x, w1, b1, wa, atr, w2)


def _tc_eaw_body(ea_r, wb_r, out_r):
    out_r[...] = jnp.dot(ea_r[...], wb_r[...].T, preferred_element_type=jnp.float32)


def _tc_eaw(edge_attr, wb):
    EB = 4000
    return pl.pallas_call(
        _tc_eaw_body,
        grid=(E // EB,),
        in_specs=[
            pl.BlockSpec((EB, 16), lambda i: (i, 0)),
            pl.BlockSpec((H, 16), lambda i: (0, 0)),
        ],
        out_specs=pl.BlockSpec((EB, H), lambda i: (i, 0)),
        out_shape=jax.ShapeDtypeStruct((E, H), jnp.float32),
    )(edge_attr, wb)


def _tc_gru_body(hlo_r, hhi_r, hb_r, xp_r, wih_r, whh_r, bih_r, bhh_r,
                 cw_r, cas_r, cad_r, xc_r, xt_r, ss_r, sd_r):
    h = jnp.concatenate([hlo_r[...], hhi_r[...]], axis=-1) + hb_r[...][None, :]
    h = _elu(h)
    xp = xp_r[...]
    gi = jnp.dot(h, wih_r[...].T, preferred_element_type=jnp.float32) + bih_r[...][None, :]
    gh = jnp.dot(xp, whh_r[...].T, preferred_element_type=jnp.float32) + bhh_r[...][None, :]
    ir, iz, inn = jnp.split(gi, 3, axis=-1)
    hr, hz, hn = jnp.split(gh, 3, axis=-1)
    r = jax.nn.sigmoid(ir + hr)
    z = jax.nn.sigmoid(iz + hz)
    n = jnp.tanh(inn + r * hn)
    xc = jax.nn.relu((1.0 - z) * n + z * xp)
    xc_r[...] = xc
    xt = jnp.dot(xc, cw_r[...].T, preferred_element_type=jnp.float32)
    xt_r[...] = xt
    ss_r[...] = jnp.sum(xt * cas_r[...][None, :], axis=-1, keepdims=True)
    sd_r[...] = jnp.sum(xt * cad_r[...][None, :], axis=-1, keepdims=True)


def _tc_gru(hlo, hhi, hbias, xprev, wih, whh, bih, bhh, convw, att_s, att_d):
    grid = (N // NB,)
    vec = lambda i: (i,)
    blk = lambda i: (i, 0)
    cst2 = lambda i: (0, 0)
    cst1 = lambda i: (0,)
    return pl.pallas_call(
        _tc_gru_body,
        grid=grid,
        in_specs=[
            pl.BlockSpec((NB, H // 2), blk),
            pl.BlockSpec((NB, H // 2), blk),
            pl.BlockSpec((H,), cst1),
            pl.BlockSpec((NB, H), blk),
            pl.BlockSpec((3 * H, H), cst2),
            pl.BlockSpec((3 * H, H), cst2),
            pl.BlockSpec((3 * H,), cst1),
            pl.BlockSpec((3 * H,), cst1),
            pl.BlockSpec((H, H), cst2),
            pl.BlockSpec((H,), cst1),
            pl.BlockSpec((H,), cst1),
        ],
        out_specs=[
            pl.BlockSpec((NB, H), blk),
            pl.BlockSpec((NB, H), blk),
            pl.BlockSpec((NB, 1), blk),
            pl.BlockSpec((NB, 1), blk),
        ],
        out_shape=[
            jax.ShapeDtypeStruct((N, H), jnp.float32),
            jax.ShapeDtypeStruct((N, H), jnp.float32),
            jax.ShapeDtypeStruct((N, 1), jnp.float32),
            jax.ShapeDtypeStruct((N, 1), jnp.float32),
        ],
    )(hlo, hhi, hbias, xprev, wih, whh, bih, bhh, convw, att_s, att_d)


def _tc_mol_body(xf_r, xt_r, asrc_r, batch_r, mw_r, mad_r, mb_r,
                 wih_r, whh_r, bih_r, bhh_r, l2w_r, l2b_r, out_r):
    bt = batch_r[...][None, :]
    gid = lax.broadcasted_iota(jnp.int32, (G, 1), 0)
    A = (bt == gid).astype(jnp.float32)
    xf = xf_r[...]
    xt = xt_r[...]
    a_src = asrc_r[...]

    out = jax.nn.relu(jnp.dot(A, xf, preferred_element_type=jnp.float32))
    wih = wih_r[...]
    whh = whh_r[...]
    bih = bih_r[...][None, :]
    bhh = bhh_r[...][None, :]
    mad = mad_r[...][None, :]
    for _ in range(2):
        ot = jnp.dot(out, mw_r[...].T, preferred_element_type=jnp.float32)
        ad = jnp.sum(ot * mad, axis=-1)
        adn = jnp.sum(ad[:, None] * A, axis=0)
        a = _leaky(a_src + adn)
        ex = jnp.exp(a)
        s3 = jnp.sum(A * ex[None, :], axis=1)
        s3n = jnp.sum(s3[:, None] * A, axis=0)
        w = ex / (s3n + 1e-16)
        hm = jnp.dot(A, xt * w[:, None], preferred_element_type=jnp.float32)
        hm = _elu(hm + mb_r[...][None, :])
        gi = jnp.dot(hm, wih.T, preferred_element_type=jnp.float32) + bih
        gh = jnp.dot(out, whh.T, preferred_element_type=jnp.float32) + bhh
        ir, iz, inn = jnp.split(gi, 3, axis=-1)
        hr, hz, hn = jnp.split(gh, 3, axis=-1)
        r = jax.nn.sigmoid(ir + hr)
        z = jax.nn.sigmoid(iz + hz)
        n = jnp.tanh(inn + r * hn)
        out = jax.nn.relu((1.0 - z) * n + z * out)
    out_r[...] = (jnp.dot(out, l2w_r[...].T, preferred_element_type=jnp.float32)
                  + l2b_r[...][None, :])


def _tc_mol(xf, xt, asrc, batch, mw, mad, mb, wih, whh, bih, bhh, l2w, l2b):
    return pl.pallas_call(
        _tc_mol_body,
        out_shape=jax.ShapeDtypeStruct((G, OUT), jnp.float32),
        compiler_params=pltpu.CompilerParams(vmem_limit_bytes=100 * 2**20),
    )(xf, xt, asrc, batch, mw, mad, mb, wih, whh, bih, bhh, l2w, l2b)


_EPT_A = E // (NCORE * NS)
_NCH_A = _EPT_A // C


def _sc_gate_alpha_body(src_h, dst_h, ai_h, xa_h, eaw_h, attl_h,
                        ex_h, sp0_h, sp1_h,
                        si, di, aig, xarows, earows, attl_v, tmp, exb, zb, s_sh,
                        sem):
    cid = lax.axis_index("c")
    t = lax.axis_index("s")
    base = (cid * NS + t) * _EPT_A

    pltpu.sync_copy(attl_h, attl_v)
    def _z(i, _):
        zb[pl.ds(i * 16, 16)] = jnp.zeros((16,), jnp.float32)
        return 0
    lax.fori_loop(0, 64, _z, 0, unroll=8)
    @pl.when(t < 10)
    def _():
        pltpu.sync_copy(zb.at[pl.ds(0, 1000)], s_sh.at[pl.ds(t * 1000, 1000)])
    plsc.subcore_barrier()

    iota16 = lax.iota(jnp.int32, 16)

    def _chunk(j, _):
        eb = base + j * C
        c1 = pltpu.async_copy(src_h.at[pl.ds(eb, C)], si, sem)
        c2 = pltpu.async_copy(dst_h.at[pl.ds(eb, C)], di, sem)
        c1.wait()
        c2.wait()
        g1 = pltpu.async_copy(ai_h.at[di], aig, sem)
        g2 = pltpu.async_copy(xa_h.at[si], xarows, sem)
        g3 = pltpu.async_copy(eaw_h.at[pl.ds(eb, C)], earows, sem)
        g1.wait()
        g2.wait()
        g3.wait()

        def _grp(g, _):
            def _edge(e16, _):
                e = g * 16 + e16
                def _fb(fb, acc):
                    v = xarows[e, pl.ds(fb * 16, 16)] + earows[e, pl.ds(fb * 16, 16)]
                    return acc + _leaky(v) * attl_v[pl.ds(fb * 16, 16)]
                acc = lax.fori_loop(0, 16, _fb, jnp.zeros((16,), jnp.float32),
                                    unroll=8)
                tmp[e16, pl.ds(0, 16)] = acc
                return 0
            lax.fori_loop(0, 16, _edge, 0)
            tot = jnp.zeros((16,), jnp.float32)
            for l in range(16):
                tot = tot + plsc.load_gather(
                    tmp, [iota16, jnp.full((16,), l, jnp.int32)])
            a16 = _leaky(tot + aig[pl.ds(g * 16, 16)])
            exb[pl.ds(g * 16, 16)] = jnp.exp(a16)
            return 0
        lax.fori_loop(0, C // 16, _grp, 0)

        pltpu.sync_copy(exb, ex_h.at[pl.ds(eb, C)])
        pltpu.sync_copy(exb, s_sh.at[di], add=True)
        return 0
    lax.fori_loop(0, _NCH_A, _chunk, 0)
    plsc.subcore_barrier()

    @pl.when(t < 10)
    def _():
        pltpu.sync_copy(s_sh.at[pl.ds(t * 1000, 1000)], zb.at[pl.ds(0, 1000)])
    @pl.when((t < 10) & (cid == 0))
    def _():
        pltpu.sync_copy(zb.at[pl.ds(0, 1000)], sp0_h.at[pl.ds(t * 1000, 1000)])
    @pl.when((t < 10) & (cid == 1))
    def _():
        pltpu.sync_copy(zb.at[pl.ds(0, 1000)], sp1_h.at[pl.ds(t * 1000, 1000)])


def _sc_gate_alpha(src, dst, ai, xa, eaw, attl):
    mesh = plsc.VectorSubcoreMesh(core_axis_name="c", subcore_axis_name="s",
                                  num_cores=NCORE, num_subcores=NS)
    kfn = pl.kernel(
        _sc_gate_alpha_body,
        out_type=[jax.ShapeDtypeStruct((E,), jnp.float32),
                  jax.ShapeDtypeStruct((N,), jnp.float32),
                  jax.ShapeDtypeStruct((N,), jnp.float32)],
        mesh=mesh,
        scratch_types=[
            pltpu.VMEM((C,), jnp.int32),
            pltpu.VMEM((C,), jnp.int32),
            pltpu.VMEM((C,), jnp.float32),
            pltpu.VMEM((C, H), jnp.float32),
            pltpu.VMEM((C, H), jnp.float32),
            pltpu.VMEM((H,), jnp.float32),
            pltpu.VMEM((16, 16), jnp.float32),
            pltpu.VMEM((C,), jnp.float32),
            pltpu.VMEM((1024,), jnp.float32),
            pltpu.VMEM_SHARED((N,), jnp.float32),
            pltpu.SemaphoreType.DMA,
        ],
        compiler_params=_SC_PARAMS,
    )
    return kfn(src, dst, ai, xa, eaw, attl)


_EPT_B = E // NS
_NCH_B = _EPT_B // C


def _sc_gate_agg_body(src_h, dst_h, ex_h, sp0_h, sp1_h, xw2p_h,
                      hlo_h, hhi_h,
                      si, di, gix, exb, s0g, s1g, wbuf, rows, h_sh, sem):
    cid = lax.axis_index("c")
    t = lax.axis_index("s")
    base = t * _EPT_B

    def _zr(i, _):
        def _zf(fb, _):
            rows[i, pl.ds(fb * 16, 16)] = jnp.zeros((16,), jnp.float32)
            return 0
        lax.fori_loop(0, 8, _zf, 0, unroll=8)
        return 0
    lax.fori_loop(0, C, _zr, 0)
    @pl.when(t < 10)
    def _():
        def _zh(j, _):
            pltpu.sync_copy(rows.at[pl.ds(0, 40)],
                            h_sh.at[pl.ds(t * 1000 + j * 40, 40)])
            return 0
        lax.fori_loop(0, 25, _zh, 0)
    plsc.subcore_barrier()

    def _chunk(j, _):
        eb = base + j * C
        c1 = pltpu.async_copy(src_h.at[pl.ds(eb, C)], si, sem)
        c2 = pltpu.async_copy(dst_h.at[pl.ds(eb, C)], di, sem)
        c3 = pltpu.async_copy(ex_h.at[pl.ds(eb, C)], exb, sem)
        c1.wait()
        c2.wait()
        c3.wait()
        g1 = pltpu.async_copy(sp0_h.at[di], s0g, sem)
        g2 = pltpu.async_copy(sp1_h.at[di], s1g, sem)
        def _gi(i, _):
            gix[pl.ds(i * 16, 16)] = si[pl.ds(i * 16, 16)] * 2 + cid
            return 0
        lax.fori_loop(0, C // 16, _gi, 0, unroll=C // 16)
        g3 = pltpu.async_copy(xw2p_h.at[gix], rows, sem)
        g1.wait()
        g2.wait()
        def _w(i, _):
            sl = pl.ds(i * 16, 16)
            wbuf[sl] = exb[sl] / (s0g[sl] + s1g[sl] + 1e-16)
            return 0
        lax.fori_loop(0, C // 16, _w, 0, unroll=C // 16)
        g3.wait()
        def _pe(e, _):
            wsp = plsc.load_gather(wbuf, [jnp.zeros((16,), jnp.int32) + e])
            def _fb(fb, _):
                rows[e, pl.ds(fb * 16, 16)] = rows[e, pl.ds(fb * 16, 16)] * wsp
                return 0
            lax.fori_loop(0, 8, _fb, 0, unroll=8)
            return 0
        lax.fori_loop(0, C, _pe, 0)
        pltpu.sync_copy(rows, h_sh.at[di], add=True)
        return 0
    lax.fori_loop(0, _NCH_B, _chunk, 0)
    plsc.subcore_barrier()

    @pl.when((t < 10) & (cid == 0))
    def _():
        def _co(j, _):
            pltpu.sync_copy(h_sh.at[pl.ds(t * 1000 + j * 40, 40)],
                            rows.at[pl.ds(0, 40)])
            pltpu.sync_copy(rows.at[pl.ds(0, 40)],
                            hlo_h.at[pl.ds(t * 1000 + j * 40, 40)])
            return 0
        lax.fori_loop(0, 25, _co, 0)
    @pl.when((t < 10) & (cid == 1))
    def _():
        def _co(j, _):
            pltpu.sync_copy(h_sh.at[pl.ds(t * 1000 + j * 40, 40)],
                            rows.at[pl.ds(0, 40)])
            pltpu.sync_copy(rows.at[pl.ds(0, 40)],
                            hhi_h.at[pl.ds(t * 1000 + j * 40, 40)])
            return 0
        lax.fori_loop(0, 25, _co, 0)


def _sc_gate_agg(src, dst, ex, sp0, sp1, xw2p):
    mesh = plsc.VectorSubcoreMesh(core_axis_name="c", subcore_axis_name="s",
                                  num_cores=NCORE, num_subcores=NS)
    kfn = pl.kernel(
        _sc_gate_agg_body,
        out_type=[jax.ShapeDtypeStruct((N, H // 2), jnp.float32),
                  jax.ShapeDtypeStruct((N, H // 2), jnp.float32)],
        mesh=mesh,
        scratch_types=[
            pltpu.VMEM((C,), jnp.int32),
            pltpu.VMEM((C,), jnp.int32),
            pltpu.VMEM((C,), jnp.int32),
            pltpu.VMEM((C,), jnp.float32),
            pltpu.VMEM((C,), jnp.float32),
            pltpu.VMEM((C,), jnp.float32),
            pltpu.VMEM((C,), jnp.float32),
            pltpu.VMEM((C, H // 2), jnp.float32),
            pltpu.VMEM_SHARED((N, H // 2), jnp.float32),
            pltpu.SemaphoreType.DMA,
        ],
        compiler_params=_SC_PARAMS,
    )
    return kfn(src, dst, ex, sp0, sp1, xw2p)


def _sc_gat_body(src_h, dst_h, ss_h, sd_h, xtp_h, hlo_h, hhi_h,
                 si, di, gix, ssg, sdg, sv, exb, wbuf, rows, zb, s_sh, h_sh,
                 sem):
    cid = lax.axis_index("c")
    t = lax.axis_index("s")
    base = t * _EPT_B

    def _z(i, _):
        zb[pl.ds(i * 16, 16)] = jnp.zeros((16,), jnp.float32)
        return 0
    lax.fori_loop(0, 64, _z, 0, unroll=8)
    def _zr(i, _):
        def _zf(fb, _):
            rows[i, pl.ds(fb * 16, 16)] = jnp.zeros((16,), jnp.float32)
            return 0
        lax.fori_loop(0, 8, _zf, 0, unroll=8)
        return 0
    lax.fori_loop(0, C, _zr, 0)
    @pl.when(t < 10)
    def _():
        pltpu.sync_copy(zb.at[pl.ds(0, 1000)], s_sh.at[pl.ds(t * 1000, 1000)])
        def _zh(j, _):
            pltpu.sync_copy(rows.at[pl.ds(0, 40)],
                            h_sh.at[pl.ds(t * 1000 + j * 40, 40)])
            return 0
        lax.fori_loop(0, 25, _zh, 0)
    plsc.subcore_barrier()

    def _chunk1(j, _):
        eb = base + j * C
        c1 = pltpu.async_copy(src_h.at[pl.ds(eb, C)], si, sem)
        c2 = pltpu.async_copy(dst_h.at[pl.ds(eb, C)], di, sem)
        c1.wait()
        c2.wait()
        g1 = pltpu.async_copy(ss_h.at[si], ssg, sem)
        g2 = pltpu.async_copy(sd_h.at[di], sdg, sem)
        g1.wait()
        g2.wait()
        def _grp(i, _):
            sl = pl.ds(i * 16, 16)
            exb[sl] = jnp.exp(_leaky(ssg[sl] + sdg[sl]))
            return 0
        lax.fori_loop(0, C // 16, _grp, 0, unroll=C // 16)
        pltpu.sync_copy(exb, s_sh.at[di], add=True)
        return 0
    lax.fori_loop(0, _NCH_B, _chunk1, 0)
    plsc.subcore_barrier()
    pltpu.sync_copy(s_sh, sv)

    def _chunk2(j, _):
        eb = base + j * C
        c1 = pltpu.async_copy(src_h.at[pl.ds(eb, C)], si, sem)
        c2 = pltpu.async_copy(dst_h.at[pl.ds(eb, C)], di, sem)
        c1.wait()
        c2.wait()
        g1 = pltpu.async_copy(ss_h.at[si], ssg, sem)
        g2 = pltpu.async_copy(sd_h.at[di], sdg, sem)
        def _gi(i, _):
            gix[pl.ds(i * 16, 16)] = si[pl.ds(i * 16, 16)] * 2 + cid
            return 0
        lax.fori_loop(0, C // 16, _gi, 0, unroll=C // 16)
        g4 = pltpu.async_copy(xtp_h.at[gix], rows, sem)
        g1.wait()
        g2.wait()
        def _w(i, _):
            sl = pl.ds(i * 16, 16)
            ex16 = jnp.exp(_leaky(ssg[sl] + sdg[sl]))
            sgath = plsc.load_gather(sv, [di[sl]])
            wbuf[sl] = ex16 / (sgath + 1e-16)
            return 0
        lax.fori_loop(0, C // 16, _w, 0, unroll=C // 16)
        g4.wait()
        def _pe(e, _):
            wsp = plsc.load_gather(wbuf, [jnp.zeros((16,), jnp.int32) + e])
            def _fb(fb, _):
                rows[e, pl.ds(fb * 16, 16)] = rows[e, pl.ds(fb * 16, 16)] * wsp
                return 0
            lax.fori_loop(0, 8, _fb, 0, unroll=8)
            return 0
        lax.fori_loop(0, C, _pe, 0)
        pltpu.sync_copy(rows, h_sh.at[di], add=True)
        return 0
    lax.fori_loop(0, _NCH_B, _chunk2, 0)
    plsc.subcore_barrier()

    @pl.when((t < 10) & (cid == 0))
    def _():
        def _co(j, _):
            pltpu.sync_copy(h_sh.at[pl.ds(t * 1000 + j * 40, 40)],
                            rows.at[pl.ds(0, 40)])
            pltpu.sync_copy(rows.at[pl.ds(0, 40)],
                            hlo_h.at[pl.ds(t * 1000 + j * 40, 40)])
            return 0
        lax.fori_loop(0, 25, _co, 0)
    @pl.when((t < 10) & (cid == 1))
    def _():
        def _co(j, _):
            pltpu.sync_copy(h_sh.at[pl.ds(t * 1000 + j * 40, 40)],
                            rows.at[pl.ds(0, 40)])
            pltpu.sync_copy(rows.at[pl.ds(0, 40)],
                            hhi_h.at[pl.ds(t * 1000 + j * 40, 40)])
            return 0
        lax.fori_loop(0, 25, _co, 0)


def _sc_gat(src, dst, ss, sd, xtp):
    mesh = plsc.VectorSubcoreMesh(core_axis_name="c", subcore_axis_name="s",
                                  num_cores=NCORE, num_subcores=NS)
    kfn = pl.kernel(
        _sc_gat_body,
        out_type=[jax.ShapeDtypeStruct((N, H // 2), jnp.float32),
                  jax.ShapeDtypeStruct((N, H // 2), jnp.float32)],
        mesh=mesh,
        scratch_types=[
            pltpu.VMEM((C,), jnp.int32),
            pltpu.VMEM((C,), jnp.int32),
            pltpu.VMEM((C,), jnp.int32),
            pltpu.VMEM((C,), jnp.float32),
            pltpu.VMEM((C,), jnp.float32),
            pltpu.VMEM((N,), jnp.float32),
            pltpu.VMEM((C,), jnp.float32),
            pltpu.VMEM((C,), jnp.float32),
            pltpu.VMEM((C, H // 2), jnp.float32),
            pltpu.VMEM((1024,), jnp.float32),
            pltpu.VMEM_SHARED((N,), jnp.float32),
            pltpu.VMEM_SHARED((N, H // 2), jnp.float32),
            pltpu.SemaphoreType.DMA,
        ],
        compiler_params=_SC_PARAMS,
    )
    return kfn(src, dst, ss, sd, xtp)


_USE_SC = {"alpha": True, "agg": True, "gat": True}


def _xla_gate_alpha(src, dst, ai, xa, eaw, attl):
    aj = (_leaky(xa[src] + eaw) * attl).sum(-1)
    a = _leaky(aj + ai[dst])
    ex = jnp.exp(a)
    s = jax.ops.segment_sum(ex, dst, num_segments=N)
    return ex, s, jnp.zeros((N,), jnp.float32)


def _xla_gate_agg(src, dst, ex, sp0, sp1, xw2p):
    s = sp0 + sp1
    w = ex / (s[dst] + 1e-16)
    xw2 = xw2p.reshape(N, H)
    h = jax.ops.segment_sum(xw2[src] * w[:, None], dst, num_segments=N)
    return h[:, :H // 2], h[:, H // 2:]


def _xla_gat(src, dst, ss, sd, xtp):
    a = _leaky(ss[src] + sd[dst])
    ex = jnp.exp(a)
    s = jax.ops.segment_sum(ex, dst, num_segments=N)
    w = ex / (s[dst] + 1e-16)
    xt = xtp.reshape(N, H)
    h = jax.ops.segment_sum(xt[src] * w[:, None], dst, num_segments=N)
    return h[:, :H // 2], h[:, H // 2:]


def kernel(x, edge_index, edge_attr, batch, params):
    p = params
    src = edge_index[0]
    dst = edge_index[1]

    w1a = p['g_lin1_W'][:, :H]
    w1b = p['g_lin1_W'][:, H:]

    x1, xa, xw2, ai = _tc_pre(x, p['lin1_W'], p['lin1_b'], w1a,
                              p['g_att_r'], p['g_lin2_W'])
    ai = ai.reshape(N)
    eaw = _tc_eaw(edge_attr, w1b)

    f_alpha = _sc_gate_alpha if _USE_SC["alpha"] else _xla_gate_alpha
    f_agg = _sc_gate_agg if _USE_SC["agg"] else _xla_gate_agg
    f_gat = _sc_gat if _USE_SC["gat"] else _xla_gat
    ex, sp0, sp1 = f_alpha(src, dst, ai, xa, eaw, p['g_att_l'])
    hlo, hhi = f_agg(src, dst, ex, sp0, sp1,
                     xw2.reshape(2 * N, H // 2))

    xc, xt, ss, sd = _tc_gru(hlo, hhi, p['g_bias'], x1,
                             p['gru0_Wih'], p['gru0_Whh'],
                             p['gru0_bih'], p['gru0_bhh'],
                             p['conv1_W'], p['conv1_att_src'],
                             p['conv1_att_dst'])
    ss, sd = ss.reshape(N), sd.reshape(N)

    hlo, hhi = f_gat(src, dst, ss, sd, xt.reshape(2 * N, H // 2))
    xc, xt, ss, sd = _tc_gru(hlo, hhi, p['conv1_bias'], xc,
                             p['gru1_Wih'], p['gru1_Whh'],
                             p['gru1_bih'], p['gru1_bhh'],
                             p['conv2_W'], p['conv2_att_src'],
                             p['conv2_att_dst'])
    ss, sd = ss.reshape(N), sd.reshape(N)

    hlo, hhi = f_gat(src, dst, ss, sd, xt.reshape(2 * N, H // 2))
    xf, xtm, asrc, _ = _tc_gru(hlo, hhi, p['conv2_bias'], xc,
                               p['gru2_Wih'], p['gru2_Whh'],
                               p['gru2_bih'], p['gru2_bhh'],
                               p['mol_W'], p['mol_att_src'],
                               p['mol_att_dst'])
    asrc = asrc.reshape(N)

    return _tc_mol(xf, xtm, asrc, batch, p['mol_W'], p['mol_att_dst'],
                   p['mol_bias'], p['mol_gru_Wih'], p['mol_gru_Whh'],
                   p['mol_gru_bih'], p['mol_gru_bhh'],
                   p['lin2_W'], p['lin2_b'])


# speedup vs baseline: 9.2093x; 1.2606x over previous
import functools
import math

import jax
import jax.numpy as jnp
from jax import lax
from jax.experimental import pallas as pl
from jax.experimental.pallas import tpu as pltpu
from jax.experimental.pallas import tpu_sc as plsc

N = 10000
E = 320000
G = 64
IN = 128
H = 256
OUT = 64
NS = 16
NCORE = 2
C = 80
NB = 1000

_SC_PARAMS = pltpu.CompilerParams(needs_layout_passes=False)


def _leaky(v):
    return jnp.where(v > 0, v, 0.01 * v)


def _elu(v):
    return jnp.where(v > 0, v, jnp.exp(jnp.minimum(v, 0.0)) - 1.0)


def _tc_pre_body(x_r, w1_r, b1_r, wa_r, atr_r, w2_r, x1_r, xa_r, xw2_r, ai_r):
    x1 = _leaky(jnp.dot(x_r[...], w1_r[...].T, preferred_element_type=jnp.float32)
                + b1_r[...][None, :])
    x1_r[...] = x1
    xa_r[...] = jnp.dot(x1, wa_r[...].T, preferred_element_type=jnp.float32)
    xw2_r[...] = jnp.dot(x1, w2_r[...].T, preferred_element_type=jnp.float32)
    ai_r[...] = jnp.sum(x1 * atr_r[...][None, :], axis=-1, keepdims=True)


def _tc_pre(x, w1, b1, wa, atr, w2):
    grid = (N // NB,)
    return pl.pallas_call(
        _tc_pre_body,
        grid=grid,
        in_specs=[
            pl.BlockSpec((NB, IN), lambda i: (i, 0)),
            pl.BlockSpec((H, IN), lambda i: (0, 0)),
            pl.BlockSpec((H,), lambda i: (0,)),
            pl.BlockSpec((H, H), lambda i: (0, 0)),
            pl.BlockSpec((H,), lambda i: (0,)),
            pl.BlockSpec((H, H), lambda i: (0, 0)),
        ],
        out_specs=[
            pl.BlockSpec((NB, H), lambda i: (i, 0)),
            pl.BlockSpec((NB, H), lambda i: (i, 0)),
            pl.BlockSpec((NB, H), lambda i: (i, 0)),
            pl.BlockSpec((NB, 1), lambda i: (i, 0)),
        ],
        out_shape=[
            jax.ShapeDtypeStruct((N, H), jnp.float32),
            jax.ShapeDtypeStruct((N, H), jnp.float32),
            jax.ShapeDtypeStruct((N, H), jnp.float32),
            jax.ShapeDtypeStruct((N, 1), jnp.float32),
        ],
    )(x, w1, b1, wa, atr, w2)


def _tc_eaw_body(ea_r, wb_r, out_r):
    out_r[...] = jnp.dot(ea_r[...], wb_r[...].T, preferred_element_type=jnp.float32)


def _tc_eaw(edge_attr, wb):
    EB = 4000
    return pl.pallas_call(
        _tc_eaw_body,
        grid=(E // EB,),
        in_specs=[
            pl.BlockSpec((EB, 16), lambda i: (i, 0)),
            pl.BlockSpec((H, 16), lambda i: (0, 0)),
        ],
        out_specs=pl.BlockSpec((EB, H), lambda i: (i, 0)),
        out_shape=jax.ShapeDtypeStruct((E, H), jnp.float32),
    )(edge_attr, wb)


def _tc_gru_body(hlo_r, hhi_r, hb_r, xp_r, wih_r, whh_r, bih_r, bhh_r,
                 cw_r, cas_r, cad_r, xc_r, xt_r, ss_r, sd_r):
    h = jnp.concatenate([hlo_r[...], hhi_r[...]], axis=-1) + hb_r[...][None, :]
    h = _elu(h)
    xp = xp_r[...]
    gi = jnp.dot(h, wih_r[...].T, preferred_element_type=jnp.float32) + bih_r[...][None, :]
    gh = jnp.dot(xp, whh_r[...].T, preferred_element_type=jnp.float32) + bhh_r[...][None, :]
    ir, iz, inn = jnp.split(gi, 3, axis=-1)
    hr, hz, hn = jnp.split(gh, 3, axis=-1)
    r = jax.nn.sigmoid(ir + hr)
    z = jax.nn.sigmoid(iz + hz)
    n = jnp.tanh(inn + r * hn)
    xc = jax.nn.relu((1.0 - z) * n + z * xp)
    xc_r[...] = xc
    xt = jnp.dot(xc, cw_r[...].T, preferred_element_type=jnp.float32)
    xt_r[...] = xt
    ss_r[...] = jnp.sum(xt * cas_r[...][None, :], axis=-1, keepdims=True)
    sd_r[...] = jnp.sum(xt * cad_r[...][None, :], axis=-1, keepdims=True)


def _tc_gru(hlo, hhi, hbias, xprev, wih, whh, bih, bhh, convw, att_s, att_d):
    grid = (N // NB,)
    vec = lambda i: (i,)
    blk = lambda i: (i, 0)
    cst2 = lambda i: (0, 0)
    cst1 = lambda i: (0,)
    return pl.pallas_call(
        _tc_gru_body,
        grid=grid,
        in_specs=[
            pl.BlockSpec((NB, H // 2), blk),
            pl.BlockSpec((NB, H // 2), blk),
            pl.BlockSpec((H,), cst1),
            pl.BlockSpec((NB, H), blk),
            pl.BlockSpec((3 * H, H), cst2),
            pl.BlockSpec((3 * H, H), cst2),
            pl.BlockSpec((3 * H,), cst1),
            pl.BlockSpec((3 * H,), cst1),
            pl.BlockSpec((H, H), cst2),
            pl.BlockSpec((H,), cst1),
            pl.BlockSpec((H,), cst1),
        ],
        out_specs=[
            pl.BlockSpec((NB, H), blk),
            pl.BlockSpec((NB, H), blk),
            pl.BlockSpec((NB, 1), blk),
            pl.BlockSpec((NB, 1), blk),
        ],
        out_shape=[
            jax.ShapeDtypeStruct((N, H), jnp.float32),
            jax.ShapeDtypeStruct((N, H), jnp.float32),
            jax.ShapeDtypeStruct((N, 1), jnp.float32),
            jax.ShapeDtypeStruct((N, 1), jnp.float32),
        ],
    )(hlo, hhi, hbias, xprev, wih, whh, bih, bhh, convw, att_s, att_d)


def _tc_mol_body(xf_r, xt_r, asrc_r, batch_r, mw_r, mad_r, mb_r,
                 wih_r, whh_r, bih_r, bhh_r, l2w_r, l2b_r, out_r):
    bt = batch_r[...][None, :]
    gid = lax.broadcasted_iota(jnp.int32, (G, 1), 0)
    A = (bt == gid).astype(jnp.float32)
    xf = xf_r[...]
    xt = xt_r[...]
    a_src = asrc_r[...]

    out = jax.nn.relu(jnp.dot(A, xf, preferred_element_type=jnp.float32))
    wih = wih_r[...]
    whh = whh_r[...]
    bih = bih_r[...][None, :]
    bhh = bhh_r[...][None, :]
    mad = mad_r[...][None, :]
    for _ in range(2):
        ot = jnp.dot(out, mw_r[...].T, preferred_element_type=jnp.float32)
        ad = jnp.sum(ot * mad, axis=-1)
        adn = jnp.sum(ad[:, None] * A, axis=0)
        a = _leaky(a_src + adn)
        ex = jnp.exp(a)
        s3 = jnp.sum(A * ex[None, :], axis=1)
        s3n = jnp.sum(s3[:, None] * A, axis=0)
        w = ex / (s3n + 1e-16)
        hm = jnp.dot(A, xt * w[:, None], preferred_element_type=jnp.float32)
        hm = _elu(hm + mb_r[...][None, :])
        gi = jnp.dot(hm, wih.T, preferred_element_type=jnp.float32) + bih
        gh = jnp.dot(out, whh.T, preferred_element_type=jnp.float32) + bhh
        ir, iz, inn = jnp.split(gi, 3, axis=-1)
        hr, hz, hn = jnp.split(gh, 3, axis=-1)
        r = jax.nn.sigmoid(ir + hr)
        z = jax.nn.sigmoid(iz + hz)
        n = jnp.tanh(inn + r * hn)
        out = jax.nn.relu((1.0 - z) * n + z * out)
    out_r[...] = (jnp.dot(out, l2w_r[...].T, preferred_element_type=jnp.float32)
                  + l2b_r[...][None, :])


def _tc_mol(xf, xt, asrc, batch, mw, mad, mb, wih, whh, bih, bhh, l2w, l2b):
    return pl.pallas_call(
        _tc_mol_body,
        out_shape=jax.ShapeDtypeStruct((G, OUT), jnp.float32),
        compiler_params=pltpu.CompilerParams(vmem_limit_bytes=100 * 2**20),
    )(xf, xt, asrc, batch, mw, mad, mb, wih, whh, bih, bhh, l2w, l2b)


_EPT_A = E // (NCORE * NS)
_NCH_A = _EPT_A // C


def _sc_gate_alpha_body(src_h, dst_h, ai_h, xa_h, eaw_h, attl_h,
                        ex_h, sp0_h, sp1_h,
                        si, di, aig, xarows, earows, attl_v, tmp, exb, zb, s_sh,
                        sem):
    cid = lax.axis_index("c")
    t = lax.axis_index("s")
    base = (cid * NS + t) * _EPT_A

    pltpu.sync_copy(attl_h, attl_v)
    def _z(i, _):
        zb[pl.ds(i * 16, 16)] = jnp.zeros((16,), jnp.float32)
        return 0
    lax.fori_loop(0, 64, _z, 0, unroll=8)
    @pl.when(t < 10)
    def _():
        pltpu.sync_copy(zb.at[pl.ds(0, 1000)], s_sh.at[pl.ds(t * 1000, 1000)])
    plsc.subcore_barrier()

    iota16 = lax.iota(jnp.int32, 16)

    def _chunk(j, _):
        eb = base + j * C
        c1 = pltpu.async_copy(src_h.at[pl.ds(eb, C)], si, sem)
        c2 = pltpu.async_copy(dst_h.at[pl.ds(eb, C)], di, sem)
        c1.wait()
        c2.wait()
        g1 = pltpu.async_copy(ai_h.at[di], aig, sem)
        g2 = pltpu.async_copy(xa_h.at[si], xarows, sem)
        g3 = pltpu.async_copy(eaw_h.at[pl.ds(eb, C)], earows, sem)
        g1.wait()
        g2.wait()
        g3.wait()

        def _grp(g, _):
            def _edge(e16, _):
                e = g * 16 + e16
                def _fb(fb, acc):
                    v = xarows[e, pl.ds(fb * 16, 16)] + earows[e, pl.ds(fb * 16, 16)]
                    return acc + _leaky(v) * attl_v[pl.ds(fb * 16, 16)]
                acc = lax.fori_loop(0, 16, _fb, jnp.zeros((16,), jnp.float32),
                                    unroll=8)
                tmp[e16, pl.ds(0, 16)] = acc
                return 0
            lax.fori_loop(0, 16, _edge, 0)
            tot = jnp.zeros((16,), jnp.float32)
            for l in range(16):
                tot = tot + plsc.load_gather(
                    tmp, [iota16, jnp.full((16,), l, jnp.int32)])
            a16 = _leaky(tot + aig[pl.ds(g * 16, 16)])
            exb[pl.ds(g * 16, 16)] = jnp.exp(a16)
            return 0
        lax.fori_loop(0, C // 16, _grp, 0)

        pltpu.sync_copy(exb, ex_h.at[pl.ds(eb, C)])
        pltpu.sync_copy(exb, s_sh.at[di], add=True)
        return 0
    lax.fori_loop(0, _NCH_A, _chunk, 0)
    plsc.subcore_barrier()

    @pl.when(t < 10)
    def _():
        pltpu.sync_copy(s_sh.at[pl.ds(t * 1000, 1000)], zb.at[pl.ds(0, 1000)])
    @pl.when((t < 10) & (cid == 0))
    def _():
        pltpu.sync_copy(zb.at[pl.ds(0, 1000)], sp0_h.at[pl.ds(t * 1000, 1000)])
    @pl.when((t < 10) & (cid == 1))
    def _():
        pltpu.sync_copy(zb.at[pl.ds(0, 1000)], sp1_h.at[pl.ds(t * 1000, 1000)])


def _sc_gate_alpha(src, dst, ai, xa, eaw, attl):
    mesh = plsc.VectorSubcoreMesh(core_axis_name="c", subcore_axis_name="s",
                                  num_cores=NCORE, num_subcores=NS)
    kfn = pl.kernel(
        _sc_gate_alpha_body,
        out_type=[jax.ShapeDtypeStruct((E,), jnp.float32),
                  jax.ShapeDtypeStruct((N,), jnp.float32),
                  jax.ShapeDtypeStruct((N,), jnp.float32)],
        mesh=mesh,
        scratch_types=[
            pltpu.VMEM((C,), jnp.int32),
            pltpu.VMEM((C,), jnp.int32),
            pltpu.VMEM((C,), jnp.float32),
            pltpu.VMEM((C, H), jnp.float32),
            pltpu.VMEM((C, H), jnp.float32),
            pltpu.VMEM((H,), jnp.float32),
            pltpu.VMEM((16, 16), jnp.float32),
            pltpu.VMEM((C,), jnp.float32),
            pltpu.VMEM((1024,), jnp.float32),
            pltpu.VMEM_SHARED((N,), jnp.float32),
            pltpu.SemaphoreType.DMA,
        ],
        compiler_params=_SC_PARAMS,
    )
    return kfn(src, dst, ai, xa, eaw, attl)


_EPT_B = E // NS
_NCH_B = _EPT_B // C


def _sc_gate_agg_body(src_h, dst_h, ex_h, sp0_h, sp1_h, xw2p_h,
                      hlo_h, hhi_h,
                      si, di, gix, exb, s0g, s1g, wbuf, rows, h_sh, sem):
    cid = lax.axis_index("c")
    t = lax.axis_index("s")
    base = t * _EPT_B

    def _zr(i, _):
        def _zf(fb, _):
            rows[i, pl.ds(fb * 16, 16)] = jnp.zeros((16,), jnp.float32)
            return 0
        lax.fori_loop(0, 8, _zf, 0, unroll=8)
        return 0
    lax.fori_loop(0, C, _zr, 0)
    @pl.when(t < 10)
    def _():
        def _zh(j, _):
            pltpu.sync_copy(rows.at[pl.ds(0, 40)],
                            h_sh.at[pl.ds(t * 1000 + j * 40, 40)])
            return 0
        lax.fori_loop(0, 25, _zh, 0)
    plsc.subcore_barrier()

    def _chunk(j, _):
        eb = base + j * C
        c1 = pltpu.async_copy(src_h.at[pl.ds(eb, C)], si, sem)
        c2 = pltpu.async_copy(dst_h.at[pl.ds(eb, C)], di, sem)
        c3 = pltpu.async_copy(ex_h.at[pl.ds(eb, C)], exb, sem)
        c1.wait()
        c2.wait()
        c3.wait()
        g1 = pltpu.async_copy(sp0_h.at[di], s0g, sem)
        g2 = pltpu.async_copy(sp1_h.at[di], s1g, sem)
        def _gi(i, _):
            gix[pl.ds(i * 16, 16)] = si[pl.ds(i * 16, 16)] * 2 + cid
            return 0
        lax.fori_loop(0, C // 16, _gi, 0, unroll=C // 16)
        g3 = pltpu.async_copy(xw2p_h.at[gix], rows, sem)
        g1.wait()
        g2.wait()
        def _w(i, _):
            sl = pl.ds(i * 16, 16)
            wbuf[sl] = exb[sl] / (s0g[sl] + s1g[sl] + 1e-16)
            return 0
        lax.fori_loop(0, C // 16, _w, 0, unroll=C // 16)
        g3.wait()
        def _pe(e, _):
            wsp = plsc.load_gather(wbuf, [jnp.zeros((16,), jnp.int32) + e])
            def _fb(fb, _):
                rows[e, pl.ds(fb * 16, 16)] = rows[e, pl.ds(fb * 16, 16)] * wsp
                return 0
            lax.fori_loop(0, 8, _fb, 0, unroll=8)
            return 0
        lax.fori_loop(0, C, _pe, 0)
        pltpu.sync_copy(rows, h_sh.at[di], add=True)
        return 0
    lax.fori_loop(0, _NCH_B, _chunk, 0)
    plsc.subcore_barrier()

    @pl.when((t < 10) & (cid == 0))
    def _():
        def _co(j, _):
            pltpu.sync_copy(h_sh.at[pl.ds(t * 1000 + j * 40, 40)],
                            rows.at[pl.ds(0, 40)])
            pltpu.sync_copy(rows.at[pl.ds(0, 40)],
                            hlo_h.at[pl.ds(t * 1000 + j * 40, 40)])
            return 0
        lax.fori_loop(0, 25, _co, 0)
    @pl.when((t < 10) & (cid == 1))
    def _():
        def _co(j, _):
            pltpu.sync_copy(h_sh.at[pl.ds(t * 1000 + j * 40, 40)],
                            rows.at[pl.ds(0, 40)])
            pltpu.sync_copy(rows.at[pl.ds(0, 40)],
                            hhi_h.at[pl.ds(t * 1000 + j * 40, 40)])
            return 0
        lax.fori_loop(0, 25, _co, 0)


def _sc_gate_agg(src, dst, ex, sp0, sp1, xw2p):
    mesh = plsc.VectorSubcoreMesh(core_axis_name="c", subcore_axis_name="s",
                                  num_cores=NCORE, num_subcores=NS)
    kfn = pl.kernel(
        _sc_gate_agg_body,
        out_type=[jax.ShapeDtypeStruct((N, H // 2), jnp.float32),
                  jax.ShapeDtypeStruct((N, H // 2), jnp.float32)],
        mesh=mesh,
        scratch_types=[
            pltpu.VMEM((C,), jnp.int32),
            pltpu.VMEM((C,), jnp.int32),
            pltpu.VMEM((C,), jnp.int32),
            pltpu.VMEM((C,), jnp.float32),
            pltpu.VMEM((C,), jnp.float32),
            pltpu.VMEM((C,), jnp.float32),
            pltpu.VMEM((C,), jnp.float32),
            pltpu.VMEM((C, H // 2), jnp.float32),
            pltpu.VMEM_SHARED((N, H // 2), jnp.float32),
            pltpu.SemaphoreType.DMA,
        ],
        compiler_params=_SC_PARAMS,
    )
    return kfn(src, dst, ex, sp0, sp1, xw2p)


def _sc_gat_body(src_h, dst_h, ss_h, sd_h, xtp_h, hlo_h, hhi_h,
                 si0, si1, di0, di1, gix0, gix1, ssg0, ssg1, sdg0, sdg1,
                 sv, wbuf, rows0, rows1, zb, s_sh, h_sh,
                 semi, sca0, sca1, srow0, srow1, ssc0, ssc1):
    cid = lax.axis_index("c")
    t = lax.axis_index("s")
    base = t * _EPT_B
    SI = [si0, si1]
    DI = [di0, di1]
    GIX = [gix0, gix1]
    SSG = [ssg0, ssg1]
    SDG = [sdg0, sdg1]
    ROWS = [rows0, rows1]
    SCA = [sca0, sca1]
    SROW = [srow0, srow1]
    SSC = [ssc0, ssc1]
    P = _NCH_B // 2

    def _z(i, _):
        zb[pl.ds(i * 16, 16)] = jnp.zeros((16,), jnp.float32)
        return 0
    lax.fori_loop(0, 64, _z, 0, unroll=8)
    def _zr(i, _):
        def _zf(fb, _):
            rows0[i, pl.ds(fb * 16, 16)] = jnp.zeros((16,), jnp.float32)
            return 0
        lax.fori_loop(0, 8, _zf, 0, unroll=8)
        return 0
    lax.fori_loop(0, C, _zr, 0)
    @pl.when(t < 10)
    def _():
        pltpu.sync_copy(zb.at[pl.ds(0, 1000)], s_sh.at[pl.ds(t * 1000, 1000)])
        def _zh(j, _):
            pltpu.sync_copy(rows0.at[pl.ds(0, 40)],
                            h_sh.at[pl.ds(t * 1000 + j * 40, 40)])
            return 0
        lax.fori_loop(0, 25, _zh, 0)
    plsc.subcore_barrier()

    def _p1_issue(k, b):
        eb = base + k * C
        c1 = pltpu.async_copy(src_h.at[pl.ds(eb, C)], SI[b], semi)
        c2 = pltpu.async_copy(dst_h.at[pl.ds(eb, C)], DI[b], semi)
        c1.wait()
        c2.wait()
        pltpu.async_copy(ss_h.at[SI[b]], SSG[b], SCA[b])
        pltpu.async_copy(sd_h.at[DI[b]], SDG[b], SCA[b])

    def _p1_compute(b):
        pltpu.make_async_copy(ss_h.at[SI[b]], SSG[b], SCA[b]).wait()
        pltpu.make_async_copy(sd_h.at[DI[b]], SDG[b], SCA[b]).wait()
        for i in range(C // 16):
            sl = pl.ds(i * 16, 16)
            wbuf[sl] = jnp.exp(_leaky(SSG[b][sl] + SDG[b][sl]))
        pltpu.sync_copy(wbuf, s_sh.at[DI[b]], add=True)

    _p1_issue(0, 0)
    def _p1_loop(j, _):
        _p1_issue(2 * j + 1, 1)
        _p1_compute(0)
        @pl.when(j < P - 1)
        def _():
            _p1_issue(2 * j + 2, 0)
        _p1_compute(1)
        return 0
    lax.fori_loop(0, P, _p1_loop, 0)
    plsc.subcore_barrier()
    pltpu.sync_copy(s_sh, sv)

    def _p2_issue(k, b, drain):
        if drain:
            pltpu.make_async_copy(ROWS[b], h_sh.at[DI[b]], SSC[b]).wait()
        eb = base + k * C
        c1 = pltpu.async_copy(src_h.at[pl.ds(eb, C)], SI[b], semi)
        c2 = pltpu.async_copy(dst_h.at[pl.ds(eb, C)], DI[b], semi)
        c1.wait()
        c2.wait()
        pltpu.async_copy(ss_h.at[SI[b]], SSG[b], SCA[b])
        pltpu.async_copy(sd_h.at[DI[b]], SDG[b], SCA[b])
        def _gi(i, _):
            sl = pl.ds(i * 16, 16)
            GIX[b][sl] = SI[b][sl] * 2 + cid
            return 0
        lax.fori_loop(0, C // 16, _gi, 0, unroll=C // 16)
        pltpu.async_copy(xtp_h.at[GIX[b]], ROWS[b], SROW[b])

    def _p2_compute(b):
        pltpu.make_async_copy(ss_h.at[SI[b]], SSG[b], SCA[b]).wait()
        pltpu.make_async_copy(sd_h.at[DI[b]], SDG[b], SCA[b]).wait()
        for i in range(C // 16):
            sl = pl.ds(i * 16, 16)
            ex16 = jnp.exp(_leaky(SSG[b][sl] + SDG[b][sl]))
            sgath = plsc.load_gather(sv, [DI[b][sl]])
            wbuf[sl] = ex16 / (sgath + 1e-16)
        pltpu.make_async_copy(xtp_h.at[GIX[b]], ROWS[b], SROW[b]).wait()
        rows_b = ROWS[b]
        def _pe(e, _):
            wsp = plsc.load_gather(wbuf, [jnp.zeros((16,), jnp.int32) + e])
            def _fb(fb, _):
                rows_b[e, pl.ds(fb * 16, 16)] = rows_b[e, pl.ds(fb * 16, 16)] * wsp
                return 0
            lax.fori_loop(0, 8, _fb, 0, unroll=8)
            return 0
        lax.fori_loop(0, C, _pe, 0)
        pltpu.async_copy(rows_b, h_sh.at[DI[b]], SSC[b], add=True)

    _p2_issue(0, 0, False)
    _p2_issue(1, 1, False)
    def _p2_loop(j, _):
        _p2_compute(0)
        @pl.when(j < P - 1)
        def _():
            _p2_issue(2 * j + 2, 0, True)
        _p2_compute(1)
        @pl.when(j < P - 1)
        def _():
            _p2_issue(2 * j + 3, 1, True)
        return 0
    lax.fori_loop(0, P, _p2_loop, 0)
    pltpu.make_async_copy(ROWS[0], h_sh.at[DI[0]], SSC[0]).wait()
    pltpu.make_async_copy(ROWS[1], h_sh.at[DI[1]], SSC[1]).wait()
    plsc.subcore_barrier()

    @pl.when((t < 10) & (cid == 0))
    def _():
        def _co(j, _):
            pltpu.sync_copy(h_sh.at[pl.ds(t * 1000 + j * 40, 40)],
                            rows0.at[pl.ds(0, 40)])
            pltpu.sync_copy(rows0.at[pl.ds(0, 40)],
                            hlo_h.at[pl.ds(t * 1000 + j * 40, 40)])
            return 0
        lax.fori_loop(0, 25, _co, 0)
    @pl.when((t < 10) & (cid == 1))
    def _():
        def _co(j, _):
            pltpu.sync_copy(h_sh.at[pl.ds(t * 1000 + j * 40, 40)],
                            rows0.at[pl.ds(0, 40)])
            pltpu.sync_copy(rows0.at[pl.ds(0, 40)],
                            hhi_h.at[pl.ds(t * 1000 + j * 40, 40)])
            return 0
        lax.fori_loop(0, 25, _co, 0)


def _sc_gat(src, dst, ss, sd, xtp):
    mesh = plsc.VectorSubcoreMesh(core_axis_name="c", subcore_axis_name="s",
                                  num_cores=NCORE, num_subcores=NS)
    kfn = pl.kernel(
        _sc_gat_body,
        out_type=[jax.ShapeDtypeStruct((N, H // 2), jnp.float32),
                  jax.ShapeDtypeStruct((N, H // 2), jnp.float32)],
        mesh=mesh,
        scratch_types=[
            pltpu.VMEM((C,), jnp.int32),
            pltpu.VMEM((C,), jnp.int32),
            pltpu.VMEM((C,), jnp.int32),
            pltpu.VMEM((C,), jnp.int32),
            pltpu.VMEM((C,), jnp.int32),
            pltpu.VMEM((C,), jnp.int32),
            pltpu.VMEM((C,), jnp.float32),
            pltpu.VMEM((C,), jnp.float32),
            pltpu.VMEM((C,), jnp.float32),
            pltpu.VMEM((C,), jnp.float32),
            pltpu.VMEM((N,), jnp.float32),
            pltpu.VMEM((C,), jnp.float32),
            pltpu.VMEM((C, H // 2), jnp.float32),
            pltpu.VMEM((C, H // 2), jnp.float32),
            pltpu.VMEM((1024,), jnp.float32),
            pltpu.VMEM_SHARED((N,), jnp.float32),
            pltpu.VMEM_SHARED((N, H // 2), jnp.float32),
            pltpu.SemaphoreType.DMA,
            pltpu.SemaphoreType.DMA,
            pltpu.SemaphoreType.DMA,
            pltpu.SemaphoreType.DMA,
            pltpu.SemaphoreType.DMA,
            pltpu.SemaphoreType.DMA,
            pltpu.SemaphoreType.DMA,
        ],
        compiler_params=_SC_PARAMS,
    )
    return kfn(src, dst, ss, sd, xtp)


_USE_SC = {"alpha": True, "agg": True, "gat": True}


def _xla_gate_alpha(src, dst, ai, xa, eaw, attl):
    aj = (_leaky(xa[src] + eaw) * attl).sum(-1)
    a = _leaky(aj + ai[dst])
    ex = jnp.exp(a)
    s = jax.ops.segment_sum(ex, dst, num_segments=N)
    return ex, s, jnp.zeros((N,), jnp.float32)


def _xla_gate_agg(src, dst, ex, sp0, sp1, xw2p):
    s = sp0 + sp1
    w = ex / (s[dst] + 1e-16)
    xw2 = xw2p.reshape(N, H)
    h = jax.ops.segment_sum(xw2[src] * w[:, None], dst, num_segments=N)
    return h[:, :H // 2], h[:, H // 2:]


def _xla_gat(src, dst, ss, sd, xtp):
    a = _leaky(ss[src] + sd[dst])
    ex = jnp.exp(a)
    s = jax.ops.segment_sum(ex, dst, num_segments=N)
    w = ex / (s[dst] + 1e-16)
    xt = xtp.reshape(N, H)
    h = jax.ops.segment_sum(xt[src] * w[:, None], dst, num_segments=N)
    return h[:, :H // 2], h[:, H // 2:]


def kernel(x, edge_index, edge_attr, batch, params):
    p = params
    src = edge_index[0]
    dst = edge_index[1]

    w1a = p['g_lin1_W'][:, :H]
    w1b = p['g_lin1_W'][:, H:]

    x1, xa, xw2, ai = _tc_pre(x, p['lin1_W'], p['lin1_b'], w1a,
                              p['g_att_r'], p['g_lin2_W'])
    ai = ai.reshape(N)
    eaw = _tc_eaw(edge_attr, w1b)

    f_alpha = _sc_gate_alpha if _USE_SC["alpha"] else _xla_gate_alpha
    f_agg = _sc_gate_agg if _USE_SC["agg"] else _xla_gate_agg
    f_gat = _sc_gat if _USE_SC["gat"] else _xla_gat
    ex, sp0, sp1 = f_alpha(src, dst, ai, xa, eaw, p['g_att_l'])
    hlo, hhi = f_agg(src, dst, ex, sp0, sp1,
                     xw2.reshape(2 * N, H // 2))

    xc, xt, ss, sd = _tc_gru(hlo, hhi, p['g_bias'], x1,
                             p['gru0_Wih'], p['gru0_Whh'],
                             p['gru0_bih'], p['gru0_bhh'],
                             p['conv1_W'], p['conv1_att_src'],
                             p['conv1_att_dst'])
    ss, sd = ss.reshape(N), sd.reshape(N)

    hlo, hhi = f_gat(src, dst, ss, sd, xt.reshape(2 * N, H // 2))
    xc, xt, ss, sd = _tc_gru(hlo, hhi, p['conv1_bias'], xc,
                             p['gru1_Wih'], p['gru1_Whh'],
                             p['gru1_bih'], p['gru1_bhh'],
                             p['conv2_W'], p['conv2_att_src'],
                             p['conv2_att_dst'])
    ss, sd = ss.reshape(N), sd.reshape(N)

    hlo, hhi = f_gat(src, dst, ss, sd, xt.reshape(2 * N, H // 2))
    xf, xtm, asrc, _ = _tc_gru(hlo, hhi, p['conv2_bias'], xc,
                               p['gru2_Wih'], p['gru2_Whh'],
                               p['gru2_bih'], p['gru2_bhh'],
                               p['mol_W'], p['mol_att_src'],
                               p['mol_att_dst'])
    asrc = asrc.reshape(N)

    return _tc_mol(xf, xtm, asrc, batch, p['mol_W'], p['mol_att_dst'],
                   p['mol_bias'], p['mol_gru_Wih'], p['mol_gru_Whh'],
                   p['mol_gru_bih'], p['mol_gru_bhh'],
                   p['lin2_W'], p['lin2_b'])
